# baseline (speedup 1.0000x reference)
import functools

import jax
import jax.numpy as jnp
from jax import lax
from jax.experimental import pallas as pl
from jax.experimental.pallas import tpu as pltpu
from jax.experimental.pallas import tpu_sc as plsc

NC = 2
NS = 16
NW = NC * NS
BW = 128
DP = 128
L = 16


@functools.lru_cache(maxsize=None)
def _build(b, s, d):
    nh = s // 2

    mesh = plsc.VectorSubcoreMesh(core_axis_name="c", subcore_axis_name="s")

    @functools.partial(
        pl.kernel,
        out_type=jax.ShapeDtypeStruct((s, d, b), jnp.float32),
        mesh=mesh,
        scratch_types=[
            pltpu.VMEM((s, BW), jnp.int32),
            pltpu.VMEM((2, BW, DP), jnp.float32),
            pltpu.VMEM((2, d, BW), jnp.float32),
            pltpu.SemaphoreType.DMA,
            pltpu.SemaphoreType.DMA,
        ],
        compiler_params=pltpu.CompilerParams(
            use_tc_tiling_on_sc=True, needs_layout_passes=False),
    )
    def k(idsT_hbm, table_hbm, out_hbm, idx_v, bufA, bufT, gsem, wsem):
        wid = lax.axis_index("s") * NC + lax.axis_index("c")
        b0 = wid * BW
        pltpu.sync_copy(idsT_hbm.at[:, pl.ds(b0, BW)], idx_v)

        iot = lax.iota(jnp.int32, L)
        rows = [iot + (g * L) for g in range(BW // L)]

        def fire_gather(j, c):
            pltpu.async_copy(table_hbm.at[idx_v.at[j]], bufA.at[c], gsem)

        def drain_g():
            pltpu.make_async_copy(
                table_hbm.at[pl.ds(0, BW)], bufA.at[0], gsem).wait()

        def drain_w():
            pltpu.make_async_copy(
                out_hbm.at[0, :, pl.ds(0, BW)], bufT.at[0], wsem).wait()

        def transpose(c):
            for d_ in range(d):
                col = iot * 0 + d_
                for g in range(BW // L):
                    v = plsc.load_gather(bufA.at[c], [rows[g], col])
                    bufT[c, d_, pl.ds(g * L, L)] = v

        def fire_write(j, c):
            pltpu.async_copy(bufT.at[c], out_hbm.at[j, :, pl.ds(b0, BW)], wsem)

        fire_gather(0, 0)

        def body(h, carry):
            for c in (0, 1):
                j = 2 * h + c

                @pl.when(j + 1 < s)
                def _():
                    fire_gather(j + 1, 1 - c)

                drain_g()

                @pl.when(h > 0)
                def _():
                    drain_w()

                transpose(c)
                fire_write(j, c)
            return carry

        lax.fori_loop(0, nh, body, 0)
        drain_w()
        drain_w()

    return k


def kernel(input_ids, table):
    b, s = input_ids.shape
    v, d = table.shape
    idsT = input_ids.astype(jnp.int32).T
    table_p = jnp.pad(table, ((0, 0), (0, DP - d)))
    out_t = _build(b, s, d)(idsT, table_p)
    return out_t.transpose(2, 0, 1)


# speedup vs baseline: 1.5787x; 1.5787x over previous
import functools

import jax
import jax.numpy as jnp
from jax import lax
from jax.experimental import pallas as pl
from jax.experimental.pallas import tpu as pltpu
from jax.experimental.pallas import tpu_sc as plsc

NC = 2
NS = 16
NW = NC * NS
CW = 128
K = 4


@functools.lru_cache(maxsize=None)
def _build(n_total, d):
    per_w = n_total // NW
    ch = per_w // CW
    ng = ch // K
    nh = ng // 2

    mesh = plsc.VectorSubcoreMesh(core_axis_name="c", subcore_axis_name="s")

    @functools.partial(
        pl.kernel,
        out_type=jax.ShapeDtypeStruct((NW, ch, CW, d), jnp.float32),
        mesh=mesh,
        scratch_types=[
            pltpu.VMEM((ch, CW), jnp.int32),
            pltpu.VMEM((2, K, CW, d), jnp.float32),
            pltpu.SemaphoreType.DMA,
            pltpu.SemaphoreType.DMA,
        ],
        compiler_params=pltpu.CompilerParams(
            use_tc_tiling_on_sc=False, skip_device_barrier=True),
    )
    def k(ids_hbm, table_hbm, out_hbm, idx_v, bufs, gsem, wsem):
        wid = lax.axis_index("s") * NC + lax.axis_index("c")
        pltpu.sync_copy(ids_hbm.at[wid], idx_v)

        def fire_gathers(g, s):
            for i in range(K):
                pltpu.async_copy(
                    table_hbm.at[idx_v.at[g * K + i]], bufs.at[s, i], gsem)

        def fire_writes(g, s):
            for i in range(K):
                pltpu.async_copy(bufs.at[s, i], out_hbm.at[wid, g * K + i], wsem)

        def drain(sem, count):
            for _ in range(count):
                pltpu.make_async_copy(out_hbm.at[wid, 0], bufs.at[0, 0], sem).wait()

        fire_gathers(0, 0)

        def body(h, carry):
            g0 = 2 * h
            g1 = g0 + 1

            @pl.when(h > 0)
            def _():
                drain(wsem, K)

            fire_gathers(g1, 1)
            drain(gsem, K)
            fire_writes(g0, 0)
            drain(gsem, K)
            fire_writes(g1, 1)
            drain(wsem, K)

            @pl.when(h + 1 < nh)
            def _():
                fire_gathers(g0 + 2, 0)

            return carry

        lax.fori_loop(0, nh, body, 0)
        drain(wsem, K)

    return k


def kernel(input_ids, table):
    b, s = input_ids.shape
    v, d = table.shape
    n = b * s
    ids = input_ids.astype(jnp.int32).reshape(NW, n // NW // CW, CW)
    out = _build(n, d)(ids, table)
    return out.reshape(b, s, d)
